# baseline (speedup 1.0000x reference)
import functools

import jax
import jax.numpy as jnp
from jax import lax
from jax.experimental import pallas as pl
from jax.experimental.pallas import tpu as pltpu
from jax.experimental.pallas import tpu_sc as plsc


@functools.lru_cache(maxsize=None)
def _make_gather(B, V, D):
    info = plsc.get_sparse_core_info()
    NC, NS = info.num_cores, info.num_subcores
    NW = NC * NS
    assert B % NW == 0
    b_per_w = B // NW
    C = 1600
    assert b_per_w % C == 0
    n_chunks = b_per_w // C
    mesh = plsc.VectorSubcoreMesh(core_axis_name="c", subcore_axis_name="s")

    @functools.partial(
        pl.kernel,
        mesh=mesh,
        compiler_params=pltpu.CompilerParams(use_tc_tiling_on_sc=False),
        out_type=jax.ShapeDtypeStruct((B, D), jnp.float32),
        scratch_types=[
            pltpu.VMEM((C,), jnp.int32),
            pltpu.VMEM((C, D), jnp.float32),
            pltpu.SemaphoreType.DMA,
        ],
    )
    def gather_kernel(idx_hbm, table_hbm, out_hbm, idx_v, rows_v, sem):
        wid = lax.axis_index("s") * NC + lax.axis_index("c")
        base = wid * b_per_w

        def body(j, carry):
            off = base + j * C
            pltpu.sync_copy(idx_hbm.at[pl.ds(off, C)], idx_v)
            pltpu.async_copy(table_hbm.at[idx_v], rows_v, sem).wait()
            pltpu.sync_copy(rows_v, out_hbm.at[pl.ds(off, C)])
            return carry

        lax.fori_loop(0, n_chunks, body, 0)

    return gather_kernel


def kernel(token_ids, model_idx, table_0):
    B, S = token_ids.shape
    V, D = table_0.shape
    idx = token_ids.reshape(-1)
    out = _make_gather(idx.shape[0], V, D)(idx, table_0)
    return out.reshape(B, S, D)


# speedup vs baseline: 1.0028x; 1.0028x over previous
import functools

import jax
import jax.numpy as jnp
from jax import lax
from jax.experimental import pallas as pl
from jax.experimental.pallas import tpu as pltpu
from jax.experimental.pallas import tpu_sc as plsc


@functools.lru_cache(maxsize=None)
def _make_gather(B, V, D):
    info = plsc.get_sparse_core_info()
    NC, NS = info.num_cores, info.num_subcores
    NW = NC * NS
    assert B % NW == 0
    b_per_w = B // NW
    C = 800
    assert b_per_w % (2 * C) == 0 and C % 8 == 0
    n_chunks = b_per_w // C
    T = n_chunks // 2
    mesh = plsc.VectorSubcoreMesh(core_axis_name="c", subcore_axis_name="s")

    @functools.partial(
        pl.kernel,
        mesh=mesh,
        compiler_params=pltpu.CompilerParams(use_tc_tiling_on_sc=False),
        out_type=jax.ShapeDtypeStruct((B, D), jnp.float32),
        scratch_types=[
            pltpu.VMEM((2, C), jnp.int32),
            pltpu.VMEM((2, C, D), jnp.float32),
            pltpu.SemaphoreType.DMA((2,)),
            pltpu.SemaphoreType.DMA((2,)),
            pltpu.SemaphoreType.DMA((2,)),
        ],
    )
    def gather_kernel(idx_hbm, table_hbm, out_hbm, idx_v, rows_v, sem_i, sem_g, sem_o):
        wid = lax.axis_index("s") * NC + lax.axis_index("c")
        base = wid * b_per_w

        def idx_load(c, b):
            return pltpu.make_async_copy(
                idx_hbm.at[pl.ds(base + c * C, C)], idx_v.at[b], sem_i.at[b])

        def row_gather(b):
            return pltpu.make_async_copy(
                table_hbm.at[idx_v.at[b]], rows_v.at[b], sem_g.at[b])

        def writeback(c, b):
            return pltpu.make_async_copy(
                rows_v.at[b], out_hbm.at[pl.ds(base + c * C, C)], sem_o.at[b])

        idx_load(0, 0).start()

        def body(t, carry):
            c0 = 2 * t
            c1 = 2 * t + 1

            @pl.when(t > 0)
            def _():
                writeback(c0 - 2, 0).wait()
            idx_load(c0, 0).wait()
            row_gather(0).start()

            @pl.when(t > 0)
            def _():
                row_gather(1).wait()
                writeback(c0 - 1, 1).start()
            idx_load(c0 + 1, 1).start()

            @pl.when(t > 0)
            def _():
                writeback(c1 - 2, 1).wait()
            idx_load(c1, 1).wait()
            row_gather(1).start()

            row_gather(0).wait()
            writeback(c0, 0).start()

            @pl.when(t < T - 1)
            def _():
                idx_load(c1 + 1, 0).start()
            return carry

        lax.fori_loop(0, T, body, 0, unroll=False)

        row_gather(1).wait()
        writeback(n_chunks - 1, 1).start()
        writeback(n_chunks - 2, 0).wait()
        writeback(n_chunks - 1, 1).wait()

    return gather_kernel


def kernel(token_ids, model_idx, table_0):
    B, S = token_ids.shape
    V, D = table_0.shape
    idx = token_ids.reshape(-1)
    out = _make_gather(idx.shape[0], V, D)(idx, table_0)
    return out.reshape(B, S, D)
